# baseline (speedup 1.0000x reference)
import functools

import jax
import jax.numpy as jnp
from jax import lax
from jax.experimental import pallas as pl
from jax.experimental.pallas import tpu as pltpu
from jax.experimental.pallas import tpu_sc as plsc

N = 10000
E = 320000
D = 128
HD = D // 2

NUM_CORES = 2
NUM_SUBCORES = 16
SUB = 80
NSUB = 5
CHUNK = SUB * NSUB
NCHUNKS = E // CHUNK
CPT = E // NUM_SUBCORES // CHUNK
ROWS_MAIN = 624
MM_BLOCK = 1000


def _mm_body(x_ref, w_ref, o_ref):
    o_ref[...] = lax.dot_general(
        x_ref[...], w_ref[...], (((1,), (1,)), ((), ())),
        preferred_element_type=jnp.float32)


def _matmul_split(seq, W):
    return pl.pallas_call(
        _mm_body,
        grid=(NUM_CORES, N // MM_BLOCK),
        in_specs=[
            pl.BlockSpec((MM_BLOCK, D), lambda c, i: (i, 0)),
            pl.BlockSpec((HD, D), lambda c, i: (c, 0)),
        ],
        out_specs=pl.BlockSpec((MM_BLOCK, HD),
                               lambda c, i: (c * (N // MM_BLOCK) + i, 0)),
        out_shape=jax.ShapeDtypeStruct((NUM_CORES * N, HD), jnp.float32),
    )(seq, W)


@functools.partial(
    pl.kernel,
    mesh=plsc.VectorSubcoreMesh(core_axis_name="c", subcore_axis_name="s"),
    out_type=jax.ShapeDtypeStruct((NUM_CORES, N, HD), jnp.float32),
    compiler_params=pltpu.CompilerParams(use_tc_tiling_on_sc=False),
    scratch_types=[
        pltpu.VMEM((NSUB, SUB), jnp.int32),
        pltpu.VMEM((NSUB, SUB), jnp.int32),
        pltpu.VMEM((CHUNK // 8, D), jnp.float32),
        pltpu.VMEM((CHUNK, HD), jnp.float32),
        pltpu.VMEM_SHARED((N, HD), jnp.float32),
        pltpu.SemaphoreType.DMA,
    ],
)
def _spmm_sc(h_hbm, col_hbm, row_hbm, w_hbm, out_hbm,
             col_v, row_v, w_v, msg_v, acc, sem):
    c = lax.axis_index("c")
    s = lax.axis_index("s")
    half = ROWS_MAIN // 2

    zero16 = jnp.zeros((16,), jnp.float32)

    def _zero_row(k, carry):
        for j in range(HD // 16):
            msg_v[k, pl.ds(j * 16, 16)] = zero16
        return carry

    lax.fori_loop(0, CHUNK, _zero_row, 0)
    r0 = s * ROWS_MAIN

    @pl.when(s == NUM_SUBCORES - 1)
    def _():
        for q in range(2):
            pltpu.sync_copy(msg_v.at[pl.ds(0, 320)],
                            acc.at[pl.ds(r0 + q * 320, 320)])

    @pl.when(s != NUM_SUBCORES - 1)
    def _():
        for q in range(2):
            pltpu.sync_copy(msg_v.at[pl.ds(0, half)],
                            acc.at[pl.ds(r0 + q * half, half)])

    plsc.subcore_barrier()

    rowoff = c * N

    def _chunk(i, carry):
        ci = s * CPT + i
        pltpu.sync_copy(col_hbm.at[ci], col_v)
        pltpu.sync_copy(row_hbm.at[ci], row_v)
        pltpu.sync_copy(w_hbm.at[ci], w_v)
        def _rebase(k, inner):
            for j in range(NSUB):
                col_v[j, pl.ds(k * 16, 16)] = (
                    col_v[j, pl.ds(k * 16, 16)] + rowoff)
            return inner

        lax.fori_loop(0, SUB // 16, _rebase, 0)
        for j in range(NSUB):
            pltpu.async_copy(h_hbm.at[col_v.at[j]],
                             msg_v.at[pl.ds(j * SUB, SUB)], sem).wait()

        def _scale(k, inner):
            wk = w_v[k // 8, pl.ds((k % 8) * 16, 16)]
            for j in range(HD // 16):
                msg_v[k, pl.ds(j * 16, 16)] = msg_v[k, pl.ds(j * 16, 16)] * wk
            return inner

        lax.fori_loop(0, CHUNK, _scale, 0)
        for j in range(NSUB):
            pltpu.sync_copy(msg_v.at[pl.ds(j * SUB, SUB)],
                            acc.at[row_v.at[j]], add=True)
        return carry

    lax.fori_loop(0, CPT, _chunk, 0)
    plsc.subcore_barrier()

    def _relu_rows(nrows, k, carry):
        for j in range(HD // 16):
            v = msg_v[k, pl.ds(j * 16, 16)]
            msg_v[k, pl.ds(j * 16, 16)] = jnp.maximum(v, 0.0)
        return carry

    @pl.when(s == NUM_SUBCORES - 1)
    def _():
        for q in range(2):
            pltpu.sync_copy(acc.at[pl.ds(r0 + q * 320, 320)],
                            msg_v.at[pl.ds(0, 320)])
            lax.fori_loop(0, 320, functools.partial(_relu_rows, 320), 0)
            pltpu.sync_copy(msg_v.at[pl.ds(0, 320)],
                            out_hbm.at[c, pl.ds(r0 + q * 320, 320)])

    @pl.when(s != NUM_SUBCORES - 1)
    def _():
        for q in range(2):
            pltpu.sync_copy(acc.at[pl.ds(r0 + q * half, half)],
                            msg_v.at[pl.ds(0, half)])
            lax.fori_loop(0, half, functools.partial(_relu_rows, half), 0)
            pltpu.sync_copy(msg_v.at[pl.ds(0, half)],
                            out_hbm.at[c, pl.ds(r0 + q * half, half)])


def kernel(seq, edge_index, edge_weight, W):
    col = edge_index[1].astype(jnp.int32).reshape(NCHUNKS, NSUB, SUB)
    row = edge_index[0].astype(jnp.int32).reshape(NCHUNKS, NSUB, SUB)
    wb = jnp.repeat(edge_weight.reshape(E // 8, 8), 16,
                    axis=-1).reshape(NCHUNKS, CHUNK // 8, D)
    h2 = _matmul_split(seq, W)
    halves = _spmm_sc(h2, col, row, wb)
    return jnp.concatenate([halves[0], halves[1]], axis=1)


# speedup vs baseline: 1.4689x; 1.4689x over previous
import functools

import jax
import jax.numpy as jnp
from jax import lax
from jax.experimental import pallas as pl
from jax.experimental.pallas import tpu as pltpu
from jax.experimental.pallas import tpu_sc as plsc

N = 10000
E = 320000
D = 128

NUM_CORES = 2
NUM_SUBCORES = 16
SUB = 80
STAGE = 2 * SUB
NSTAGES = E // STAGE
NST = 62
ROWS_MAIN = 624
MM_BLOCK = 1000


def _mm_body(x_ref, w_ref, o_ref):
    o_ref[...] = lax.dot_general(
        x_ref[...], w_ref[...], (((1,), (1,)), ((), ())),
        preferred_element_type=jnp.float32)


def _matmul(seq, W):
    return pl.pallas_call(
        _mm_body,
        grid=(N // MM_BLOCK,),
        in_specs=[
            pl.BlockSpec((MM_BLOCK, D), lambda i: (i, 0)),
            pl.BlockSpec((D, D), lambda i: (0, 0)),
        ],
        out_specs=pl.BlockSpec((MM_BLOCK, D), lambda i: (i, 0)),
        out_shape=jax.ShapeDtypeStruct((N, D), jnp.float32),
    )(seq, W)


def _combine_body(a_ref, b_ref, o_ref):
    o_ref[...] = jnp.maximum(a_ref[...] + b_ref[...], 0.0)


def _combine(a, b):
    return pl.pallas_call(
        _combine_body,
        grid=(N // MM_BLOCK,),
        in_specs=[
            pl.BlockSpec((MM_BLOCK, D), lambda i: (i, 0)),
            pl.BlockSpec((MM_BLOCK, D), lambda i: (i, 0)),
        ],
        out_specs=pl.BlockSpec((MM_BLOCK, D), lambda i: (i, 0)),
        out_shape=jax.ShapeDtypeStruct((N, D), jnp.float32),
    )(a, b)


@functools.partial(
    pl.kernel,
    mesh=plsc.VectorSubcoreMesh(core_axis_name="c", subcore_axis_name="s"),
    out_type=jax.ShapeDtypeStruct((NUM_CORES, N, D), jnp.float32),
    scratch_types=[
        pltpu.VMEM((2, SUB), jnp.int32),
        pltpu.VMEM((2, SUB), jnp.int32),
        pltpu.VMEM((2, SUB), jnp.int32),
        pltpu.VMEM((2, SUB), jnp.int32),
        pltpu.VMEM((STAGE // 8, D), jnp.float32),
        pltpu.VMEM((STAGE // 8, D), jnp.float32),
        pltpu.VMEM((STAGE, D), jnp.float32),
        pltpu.VMEM((STAGE, D), jnp.float32),
        pltpu.VMEM_SHARED((N, D), jnp.float32),
        pltpu.SemaphoreType.DMA,
        pltpu.SemaphoreType.DMA,
        pltpu.SemaphoreType.DMA,
        pltpu.SemaphoreType.DMA,
        pltpu.SemaphoreType.DMA,
        pltpu.SemaphoreType.DMA,
        pltpu.SemaphoreType.DMA,
        pltpu.SemaphoreType.DMA,
        pltpu.SemaphoreType.DMA,
        pltpu.SemaphoreType.DMA,
    ],
)
def _spmm_sc(h_hbm, col_hbm, row_hbm, w_hbm, part_hbm,
             col_a, col_b, row_a, row_b, w_a, w_b, msg_a, msg_b, acc,
             sem_c0, sem_c1, sem_r0, sem_r1, sem_w0, sem_w1,
             sem_g0, sem_g1, sem_s0, sem_s1):
    cols = (col_a, col_b)
    rows = (row_a, row_b)
    ws = (w_a, w_b)
    msgs = (msg_a, msg_b)
    c = lax.axis_index("c")
    s = lax.axis_index("s")
    wid = c * NUM_SUBCORES + s
    sems_c = (sem_c0, sem_c1)
    sems_r = (sem_r0, sem_r1)
    sems_w = (sem_w0, sem_w1)
    sems_g = (sem_g0, sem_g1)
    sems_s = (sem_s0, sem_s1)

    pair = wid // 2
    within = wid % 2
    gbase = pair * 125 + within * 63

    def drain_idx(dst, sem):
        pltpu.make_async_copy(col_hbm.at[0], dst, sem).wait()

    def drain_w(slot):
        pltpu.make_async_copy(w_hbm.at[0], ws[slot], sems_w[slot]).wait()

    def drain_g(slot):
        for j in range(2):
            pltpu.make_async_copy(h_hbm.at[cols[slot].at[j]],
                                  msgs[slot].at[pl.ds(j * SUB, SUB)],
                                  sems_g[slot]).wait()

    def drain_s(slot):
        for j in range(2):
            pltpu.make_async_copy(msgs[slot].at[pl.ds(j * SUB, SUB)],
                                  acc.at[rows[slot].at[j]],
                                  sems_s[slot]).wait()

    def fire_gathers(g_unused, slot):
        for j in range(2):
            pltpu.async_copy(h_hbm.at[cols[slot].at[j]],
                             msgs[slot].at[pl.ds(j * SUB, SUB)],
                             sems_g[slot])

    def fire_scatters(slot):
        for j in range(2):
            pltpu.async_copy(msgs[slot].at[pl.ds(j * SUB, SUB)],
                             acc.at[rows[slot].at[j]], sems_s[slot],
                             add=True)

    def scale(slot):
        def _body(k, carry):
            wk = ws[slot][k // 8, pl.ds((k % 8) * 16, 16)]
            for j in range(D // 16):
                msgs[slot][k, pl.ds(j * 16, 16)] = (
                    msgs[slot][k, pl.ds(j * 16, 16)] * wk)
            return carry

        lax.fori_loop(0, STAGE, _body, 0)

    zero16 = jnp.zeros((16,), jnp.float32)

    def _zero_row(k, carry):
        for j in range(D // 16):
            msg_a[k, pl.ds(j * 16, 16)] = zero16
        return carry

    lax.fori_loop(0, STAGE, _zero_row, 0)
    r0 = s * ROWS_MAIN
    copy_sizes = (STAGE, STAGE, STAGE, 144)
    copy_sizes_last = (STAGE, STAGE, STAGE, STAGE)

    @pl.when(s == NUM_SUBCORES - 1)
    def _():
        off = 0
        for sz in copy_sizes_last:
            pltpu.sync_copy(msg_a.at[pl.ds(0, sz)],
                            acc.at[pl.ds(r0 + off, sz)])
            off += sz

    @pl.when(s != NUM_SUBCORES - 1)
    def _():
        off = 0
        for sz in copy_sizes:
            pltpu.sync_copy(msg_a.at[pl.ds(0, sz)],
                            acc.at[pl.ds(r0 + off, sz)])
            off += sz

    plsc.subcore_barrier()

    pltpu.async_copy(col_hbm.at[gbase], col_a, sem_c0)
    pltpu.async_copy(row_hbm.at[gbase], row_a, sem_r0)
    pltpu.async_copy(w_hbm.at[gbase], w_a, sem_w0)
    pltpu.async_copy(col_hbm.at[gbase + 1], col_b, sem_c1)
    pltpu.async_copy(w_hbm.at[gbase + 1], w_b, sem_w1)
    drain_idx(col_a, sem_c0)
    fire_gathers(gbase, 0)

    def _iter(t2, carry):
        for parity in range(2):
            p = parity
            q = 1 - p
            t = 2 * t2 + parity
            g = gbase + t

            def _fut1_ops():
                drain_idx(cols[q], sems_c[q])
                pltpu.async_copy(row_hbm.at[g + 1], rows[q],
                                 sems_r[q])
                fire_gathers(g + 1, q)

            def _fut2_fire_col():
                pltpu.async_copy(col_hbm.at[g + 2], cols[p], sems_c[p])

            def _fut2_fire_w():
                pltpu.async_copy(w_hbm.at[g + 2], ws[p], sems_w[p])

            if parity == 0:
                @pl.when(t2 > 0)
                def _():
                    drain_s(q)
                _fut1_ops()
            else:
                drain_s(q)

                @pl.when(t2 < 30)
                def _():
                    _fut1_ops()

            drain_g(p)

            @pl.when(t2 < 30)
            def _():
                _fut2_fire_col()

            drain_w(p)
            scale(p)
            drain_idx(rows[p], sems_r[p])
            fire_scatters(p)

            @pl.when(t2 < 30)
            def _():
                _fut2_fire_w()
        return carry

    lax.fori_loop(0, NST // 2, _iter, 0)

    drain_s(1)

    @pl.when(within == 0)
    def _():
        g = gbase + NST
        pltpu.sync_copy(col_hbm.at[g], col_a)
        pltpu.sync_copy(row_hbm.at[g], row_a)
        pltpu.sync_copy(w_hbm.at[g], w_a)
        fire_gathers(g, 0)
        drain_g(0)
        scale(0)
        for j in range(2):
            pltpu.sync_copy(msg_a.at[pl.ds(j * SUB, SUB)],
                            acc.at[row_a.at[j]], add=True)

    plsc.subcore_barrier()

    @pl.when(s == NUM_SUBCORES - 1)
    def _():
        off = 0
        for sz in copy_sizes_last:
            pltpu.sync_copy(acc.at[pl.ds(r0 + off, sz)],
                            msg_a.at[pl.ds(0, sz)])
            pltpu.sync_copy(msg_a.at[pl.ds(0, sz)],
                            part_hbm.at[c, pl.ds(r0 + off, sz)])
            off += sz

    @pl.when(s != NUM_SUBCORES - 1)
    def _():
        off = 0
        for sz in copy_sizes:
            pltpu.sync_copy(acc.at[pl.ds(r0 + off, sz)],
                            msg_a.at[pl.ds(0, sz)])
            pltpu.sync_copy(msg_a.at[pl.ds(0, sz)],
                            part_hbm.at[c, pl.ds(r0 + off, sz)])
            off += sz


def kernel(seq, edge_index, edge_weight, W):
    col = edge_index[1].astype(jnp.int32).reshape(NSTAGES, 2, SUB)
    row = edge_index[0].astype(jnp.int32).reshape(NSTAGES, 2, SUB)
    wb = jnp.repeat(edge_weight.reshape(E // 8, 8), 16,
                    axis=-1).reshape(NSTAGES, STAGE // 8, D)
    h = _matmul(seq, W)
    part = _spmm_sc(h, col, row, wb)
    return _combine(part[0], part[1])


# speedup vs baseline: 1.4830x; 1.0096x over previous
import functools

import jax
import jax.numpy as jnp
from jax import lax
from jax.experimental import pallas as pl
from jax.experimental.pallas import tpu as pltpu
from jax.experimental.pallas import tpu_sc as plsc

N = 10000
E = 320000
D = 128

NUM_CORES = 2
NUM_SUBCORES = 16
SUB = 80
STAGE = 2 * SUB
NSTAGES = E // STAGE
NST = 62
ROWS_MAIN = 624
MM_BLOCK = 1000


def _mm_body(x_ref, w_ref, o_ref):
    o_ref[...] = lax.dot_general(
        x_ref[...], w_ref[...], (((1,), (1,)), ((), ())),
        preferred_element_type=jnp.float32)


def _matmul(seq, W):
    return pl.pallas_call(
        _mm_body,
        grid=(N // MM_BLOCK,),
        in_specs=[
            pl.BlockSpec((MM_BLOCK, D), lambda i: (i, 0)),
            pl.BlockSpec((D, D), lambda i: (0, 0)),
        ],
        out_specs=pl.BlockSpec((MM_BLOCK, D), lambda i: (i, 0)),
        out_shape=jax.ShapeDtypeStruct((N, D), jnp.float32),
    )(seq, W)


def _combine_body(a_ref, b_ref, o_ref):
    o_ref[...] = jnp.maximum(a_ref[...] + b_ref[...], 0.0)


def _combine(a, b):
    return pl.pallas_call(
        _combine_body,
        grid=(N // MM_BLOCK,),
        in_specs=[
            pl.BlockSpec((MM_BLOCK, D), lambda i: (i, 0)),
            pl.BlockSpec((MM_BLOCK, D), lambda i: (i, 0)),
        ],
        out_specs=pl.BlockSpec((MM_BLOCK, D), lambda i: (i, 0)),
        out_shape=jax.ShapeDtypeStruct((N, D), jnp.float32),
    )(a, b)


@functools.partial(
    pl.kernel,
    mesh=plsc.VectorSubcoreMesh(core_axis_name="c", subcore_axis_name="s"),
    out_type=jax.ShapeDtypeStruct((NUM_CORES, N, D), jnp.float32),
    scratch_types=[
        pltpu.VMEM((2, SUB), jnp.int32),
        pltpu.VMEM((2, SUB), jnp.int32),
        pltpu.VMEM((2, SUB), jnp.int32),
        pltpu.VMEM((2, SUB), jnp.int32),
        pltpu.VMEM((STAGE // 8, D), jnp.float32),
        pltpu.VMEM((STAGE // 8, D), jnp.float32),
        pltpu.VMEM((STAGE, D), jnp.float32),
        pltpu.VMEM((STAGE, D), jnp.float32),
        pltpu.VMEM_SHARED((N, D), jnp.float32),
        pltpu.SemaphoreType.DMA,
        pltpu.SemaphoreType.DMA,
        pltpu.SemaphoreType.DMA,
        pltpu.SemaphoreType.DMA,
        pltpu.SemaphoreType.DMA,
        pltpu.SemaphoreType.DMA,
        pltpu.SemaphoreType.DMA,
        pltpu.SemaphoreType.DMA,
        pltpu.SemaphoreType.DMA,
        pltpu.SemaphoreType.DMA,
    ],
)
def _spmm_sc(h_hbm, col_hbm, row_hbm, w_hbm, part_hbm,
             col_a, col_b, row_a, row_b, w_a, w_b, msg_a, msg_b, acc,
             sem_c0, sem_c1, sem_r0, sem_r1, sem_w0, sem_w1,
             sem_g0, sem_g1, sem_s0, sem_s1):
    cols = (col_a, col_b)
    rows = (row_a, row_b)
    ws = (w_a, w_b)
    msgs = (msg_a, msg_b)
    c = lax.axis_index("c")
    s = lax.axis_index("s")
    wid = c * NUM_SUBCORES + s
    sems_c = (sem_c0, sem_c1)
    sems_r = (sem_r0, sem_r1)
    sems_w = (sem_w0, sem_w1)
    sems_g = (sem_g0, sem_g1)
    sems_s = (sem_s0, sem_s1)

    pair = wid // 2
    within = wid % 2
    gbase = pair * 125 + within * 63

    def drain_idx(dst, sem):
        pltpu.make_async_copy(col_hbm.at[0], dst, sem).wait()

    def drain_w(slot):
        pltpu.make_async_copy(w_hbm.at[0], ws[slot], sems_w[slot]).wait()

    def drain_g(slot):
        for j in range(2):
            pltpu.make_async_copy(h_hbm.at[cols[slot].at[j]],
                                  msgs[slot].at[pl.ds(j * SUB, SUB)],
                                  sems_g[slot]).wait()

    def drain_s(slot):
        for j in range(2):
            pltpu.make_async_copy(msgs[slot].at[pl.ds(j * SUB, SUB)],
                                  acc.at[rows[slot].at[j]],
                                  sems_s[slot]).wait()

    def fire_gathers(g_unused, slot):
        for j in range(2):
            pltpu.async_copy(h_hbm.at[cols[slot].at[j]],
                             msgs[slot].at[pl.ds(j * SUB, SUB)],
                             sems_g[slot])

    def fire_scatters(slot):
        for j in range(2):
            pltpu.async_copy(msgs[slot].at[pl.ds(j * SUB, SUB)],
                             acc.at[rows[slot].at[j]], sems_s[slot],
                             add=True)

    def scale(slot):
        def _body(kk, carry):
            base = kk * 8
            for t in range(8):
                wk = ws[slot][kk, pl.ds(t * 16, 16)]
                for j in range(D // 16):
                    msgs[slot][base + t, pl.ds(j * 16, 16)] = (
                        msgs[slot][base + t, pl.ds(j * 16, 16)] * wk)
            return carry

        lax.fori_loop(0, STAGE // 8, _body, 0)

    zero16 = jnp.zeros((16,), jnp.float32)

    def _zero_row(k, carry):
        for j in range(D // 16):
            msg_a[k, pl.ds(j * 16, 16)] = zero16
        return carry

    lax.fori_loop(0, STAGE, _zero_row, 0)
    r0 = s * ROWS_MAIN
    copy_sizes = (STAGE, STAGE, STAGE, 144)
    copy_sizes_last = (STAGE, STAGE, STAGE, STAGE)

    @pl.when(s == NUM_SUBCORES - 1)
    def _():
        off = 0
        for sz in copy_sizes_last:
            pltpu.sync_copy(msg_a.at[pl.ds(0, sz)],
                            acc.at[pl.ds(r0 + off, sz)])
            off += sz

    @pl.when(s != NUM_SUBCORES - 1)
    def _():
        off = 0
        for sz in copy_sizes:
            pltpu.sync_copy(msg_a.at[pl.ds(0, sz)],
                            acc.at[pl.ds(r0 + off, sz)])
            off += sz

    plsc.subcore_barrier()

    pltpu.async_copy(col_hbm.at[gbase], col_a, sem_c0)
    pltpu.async_copy(row_hbm.at[gbase], row_a, sem_r0)
    pltpu.async_copy(w_hbm.at[gbase], w_a, sem_w0)
    pltpu.async_copy(col_hbm.at[gbase + 1], col_b, sem_c1)
    pltpu.async_copy(w_hbm.at[gbase + 1], w_b, sem_w1)
    drain_idx(col_a, sem_c0)
    fire_gathers(gbase, 0)

    def _iter(t2, carry):
        for parity in range(2):
            p = parity
            q = 1 - p
            t = 2 * t2 + parity
            g = gbase + t

            def _fut1_ops():
                drain_idx(cols[q], sems_c[q])
                pltpu.async_copy(row_hbm.at[g + 1], rows[q],
                                 sems_r[q])
                fire_gathers(g + 1, q)

            def _fut2_fire_col():
                pltpu.async_copy(col_hbm.at[g + 2], cols[p], sems_c[p])

            def _fut2_fire_w():
                pltpu.async_copy(w_hbm.at[g + 2], ws[p], sems_w[p])

            if parity == 0:
                @pl.when(t2 > 0)
                def _():
                    drain_s(q)
                _fut1_ops()
            else:
                drain_s(q)

                @pl.when(t2 < 30)
                def _():
                    _fut1_ops()

            drain_g(p)

            @pl.when(t2 < 30)
            def _():
                _fut2_fire_col()

            drain_w(p)
            scale(p)
            drain_idx(rows[p], sems_r[p])
            fire_scatters(p)

            @pl.when(t2 < 30)
            def _():
                _fut2_fire_w()
        return carry

    lax.fori_loop(0, NST // 2, _iter, 0)

    drain_s(1)

    @pl.when(within == 0)
    def _():
        g = gbase + NST
        pltpu.sync_copy(col_hbm.at[g], col_a)
        pltpu.sync_copy(row_hbm.at[g], row_a)
        pltpu.sync_copy(w_hbm.at[g], w_a)
        fire_gathers(g, 0)
        drain_g(0)
        scale(0)
        for j in range(2):
            pltpu.sync_copy(msg_a.at[pl.ds(j * SUB, SUB)],
                            acc.at[row_a.at[j]], add=True)

    plsc.subcore_barrier()

    @pl.when(s == NUM_SUBCORES - 1)
    def _():
        off = 0
        for sz in copy_sizes_last:
            pltpu.sync_copy(acc.at[pl.ds(r0 + off, sz)],
                            msg_a.at[pl.ds(0, sz)])
            pltpu.sync_copy(msg_a.at[pl.ds(0, sz)],
                            part_hbm.at[c, pl.ds(r0 + off, sz)])
            off += sz

    @pl.when(s != NUM_SUBCORES - 1)
    def _():
        off = 0
        for sz in copy_sizes:
            pltpu.sync_copy(acc.at[pl.ds(r0 + off, sz)],
                            msg_a.at[pl.ds(0, sz)])
            pltpu.sync_copy(msg_a.at[pl.ds(0, sz)],
                            part_hbm.at[c, pl.ds(r0 + off, sz)])
            off += sz


def kernel(seq, edge_index, edge_weight, W):
    col = edge_index[1].astype(jnp.int32).reshape(NSTAGES, 2, SUB)
    row = edge_index[0].astype(jnp.int32).reshape(NSTAGES, 2, SUB)
    wb = jnp.repeat(edge_weight.reshape(E // 8, 8), 16,
                    axis=-1).reshape(NSTAGES, STAGE // 8, D)
    h = _matmul(seq, W)
    part = _spmm_sc(h, col, row, wb)
    return _combine(part[0], part[1])
